# baseline (speedup 1.0000x reference)
import functools

import jax
import jax.numpy as jnp
from jax.experimental import pallas as pl

NUM_EMBS = 8192
EMB_DIM = 32
BETA = 0.25
N_TOKENS = 8192
TILE = 512
GRID = N_TOKENS // TILE


def _body(x_ref, sx_ref, sw_ref, w_ref,
          idx_ref, zq_ref, hist_ref, loss_ref, perp_ref):
    step = pl.program_id(0)

    @pl.when(step == 0)
    def _init():
        hist_ref[...] = jnp.zeros_like(hist_ref)
        loss_ref[...] = jnp.zeros_like(loss_ref)
        perp_ref[...] = jnp.zeros_like(perp_ref)

    x = x_ref[...]
    w = w_ref[...]

    mm = jax.lax.dot_general(x, w, (((1,), (1,)), ((), ())),
                             preferred_element_type=jnp.float32)
    d = (sx_ref[...] + sw_ref[...]) - 2.0 * mm

    gmin = jnp.min(d, axis=1, keepdims=True)
    col = jax.lax.broadcasted_iota(jnp.int32, (TILE, NUM_EMBS), 1)
    idx = jnp.min(jnp.where(d == gmin, col, NUM_EMBS), axis=1,
                  keepdims=True)
    idx_ref[...] = idx

    oh = (col == idx).astype(jnp.float32)
    q = jax.lax.dot_general(oh, w, (((1,), (0,)), ((), ())),
                            preferred_element_type=jnp.float32)

    hist_ref[...] += jnp.sum(oh, axis=0)[None, :]
    zq_ref[...] = x + (q - x)
    loss_ref[...] += jnp.sum((q - x) ** 2)

    @pl.when(step == GRID - 1)
    def _fini():
        loss_ref[...] = (1.0 + BETA) * loss_ref[...] / (N_TOKENS * EMB_DIM)
        probs = hist_ref[...] / N_TOKENS
        ent = -jnp.sum(probs * jnp.log(probs + 1e-10))
        perp_ref[...] = jnp.exp(ent) * jnp.ones_like(perp_ref)


def kernel(z_e_x, W):
    B, C, H, Wd = z_e_x.shape
    x_flat = jnp.transpose(z_e_x, (0, 2, 3, 1)).reshape(-1, EMB_DIM)
    sx = jnp.sum(x_flat ** 2, axis=1, keepdims=True)
    sw = jnp.sum(W ** 2, axis=1)[None, :]

    idx, zq, hist, loss, perp = pl.pallas_call(
        _body,
        grid=(GRID,),
        in_specs=[
            pl.BlockSpec((TILE, EMB_DIM), lambda i: (i, 0)),
            pl.BlockSpec((TILE, 1), lambda i: (i, 0)),
            pl.BlockSpec((1, NUM_EMBS), lambda i: (0, 0)),
            pl.BlockSpec((NUM_EMBS, EMB_DIM), lambda i: (0, 0)),
        ],
        out_specs=[
            pl.BlockSpec((TILE, 1), lambda i: (i, 0)),
            pl.BlockSpec((TILE, EMB_DIM), lambda i: (i, 0)),
            pl.BlockSpec((1, NUM_EMBS), lambda i: (0, 0)),
            pl.BlockSpec((1, 1), lambda i: (0, 0)),
            pl.BlockSpec((1, 1), lambda i: (0, 0)),
        ],
        out_shape=[
            jax.ShapeDtypeStruct((N_TOKENS, 1), jnp.int32),
            jax.ShapeDtypeStruct((N_TOKENS, EMB_DIM), jnp.float32),
            jax.ShapeDtypeStruct((1, NUM_EMBS), jnp.float32),
            jax.ShapeDtypeStruct((1, 1), jnp.float32),
            jax.ShapeDtypeStruct((1, 1), jnp.float32),
        ],
    )(x_flat, sx, sw, W)

    z_q_x = jnp.transpose(zq.reshape(B, H, Wd, C), (0, 3, 1, 2))
    return (loss[0, 0], z_q_x, perp[0, 0], idx)


# speedup vs baseline: 1.2450x; 1.2450x over previous
import functools

import jax
import jax.numpy as jnp
from jax.experimental import pallas as pl

NUM_EMBS = 8192
EMB_DIM = 32
BETA = 0.25
N_TOKENS = 8192
TILE = 512
GRID = N_TOKENS // TILE


def _body(x_ref, sx_ref, sw_ref, w_ref, wneg2_ref,
          idx_ref, zq_ref, hist_ref, loss_ref, perp_ref):
    step = pl.program_id(0)

    @pl.when(step == 0)
    def _init():
        hist_ref[...] = jnp.zeros_like(hist_ref)
        loss_ref[...] = jnp.zeros_like(loss_ref)
        perp_ref[...] = jnp.zeros_like(perp_ref)

    x = x_ref[...]
    w = w_ref[...]

    mm2 = jax.lax.dot_general(x, wneg2_ref[...], (((1,), (1,)), ((), ())),
                              preferred_element_type=jnp.float32)
    d = (sx_ref[...] + sw_ref[...]) + mm2

    col = jax.lax.broadcasted_iota(jnp.int32, (TILE, NUM_EMBS), 1)
    idx = jnp.argmin(d, axis=1).astype(jnp.int32)[:, None]
    idx_ref[...] = idx

    oh = (col == idx).astype(jnp.float32)
    q = jax.lax.dot_general(oh, w, (((1,), (0,)), ((), ())),
                            preferred_element_type=jnp.float32)

    hist_ref[...] += jnp.sum(oh, axis=0)[None, :]
    zq_ref[...] = x + (q - x)
    loss_ref[...] += jnp.sum((q - x) ** 2)

    @pl.when(step == GRID - 1)
    def _fini():
        loss_ref[...] = (1.0 + BETA) * loss_ref[...] / (N_TOKENS * EMB_DIM)
        probs = hist_ref[...] / N_TOKENS
        ent = -jnp.sum(probs * jnp.log(probs + 1e-10))
        perp_ref[...] = jnp.exp(ent) * jnp.ones_like(perp_ref)


def kernel(z_e_x, W):
    B, C, H, Wd = z_e_x.shape
    x_flat = jnp.transpose(z_e_x, (0, 2, 3, 1)).reshape(-1, EMB_DIM)
    sx = jnp.sum(x_flat ** 2, axis=1, keepdims=True)
    sw = jnp.sum(W ** 2, axis=1)[None, :]

    idx, zq, hist, loss, perp = pl.pallas_call(
        _body,
        grid=(GRID,),
        in_specs=[
            pl.BlockSpec((TILE, EMB_DIM), lambda i: (i, 0)),
            pl.BlockSpec((TILE, 1), lambda i: (i, 0)),
            pl.BlockSpec((1, NUM_EMBS), lambda i: (0, 0)),
            pl.BlockSpec((NUM_EMBS, EMB_DIM), lambda i: (0, 0)),
            pl.BlockSpec((NUM_EMBS, EMB_DIM), lambda i: (0, 0)),
        ],
        out_specs=[
            pl.BlockSpec((TILE, 1), lambda i: (i, 0)),
            pl.BlockSpec((TILE, EMB_DIM), lambda i: (i, 0)),
            pl.BlockSpec((1, NUM_EMBS), lambda i: (0, 0)),
            pl.BlockSpec((1, 1), lambda i: (0, 0)),
            pl.BlockSpec((1, 1), lambda i: (0, 0)),
        ],
        out_shape=[
            jax.ShapeDtypeStruct((N_TOKENS, 1), jnp.int32),
            jax.ShapeDtypeStruct((N_TOKENS, EMB_DIM), jnp.float32),
            jax.ShapeDtypeStruct((1, NUM_EMBS), jnp.float32),
            jax.ShapeDtypeStruct((1, 1), jnp.float32),
            jax.ShapeDtypeStruct((1, 1), jnp.float32),
        ],
    )(x_flat, sx, sw, W, -2.0 * W)

    z_q_x = jnp.transpose(zq.reshape(B, H, Wd, C), (0, 3, 1, 2))
    return (loss[0, 0], z_q_x, perp[0, 0], idx)
